# Initial kernel scaffold; baseline (speedup 1.0000x reference)
import functools

import jax
import jax.numpy as jnp
from jax import lax
from jax.experimental import pallas as pl
from jax.experimental.pallas import tpu as pltpu
from jax.experimental.pallas import tpu_sc as plsc

_N = 10000
_E = 320000
_DE = 16
_H = 128
_L = 6
_OUT = 6
_SPLIT = 3
_G = 8
_EPS = 1e-5

_BLK = 2000
_NB = _N // _BLK
_EB = 2560
_NTILES = 32
_EPT = _E // _NTILES
_CH = 80
_NCH = _EPT // _CH
_RPS = _N // 16
_ZR = 125

_F32 = jnp.float32


def _edge_mlp_body(ea_ref, w_ref, b_ref, out_ref):
    out_ref[...] = (
        jnp.dot(ea_ref[...], w_ref[...], preferred_element_type=_F32)
        + b_ref[...]
    )


def _edge_mlp(edge_attr, wt, b_row):
    return pl.pallas_call(
        _edge_mlp_body,
        grid=(_E // _EB,),
        in_specs=[
            pl.BlockSpec((_EB, _DE), lambda i: (i, 0)),
            pl.BlockSpec((_DE, _H), lambda i: (0, 0)),
            pl.BlockSpec((1, _H), lambda i: (0, 0)),
        ],
        out_specs=pl.BlockSpec((_EB, _H), lambda i: (i, 0)),
        out_shape=jax.ShapeDtypeStruct((_E, _H), _F32),
    )(edge_attr, wt, b_row)


def _cnt_body(br_ref, cnt_ref):
    i = pl.program_id(0)
    oht = (br_ref[0] == lax.broadcasted_iota(_F32, (_G, 1), 0)).astype(_F32)
    part = jnp.broadcast_to(jnp.sum(oht, axis=1, keepdims=True), (_G, _H))

    @pl.when(i == 0)
    def _():
        cnt_ref[...] = jnp.zeros_like(cnt_ref)

    cnt_ref[...] += part


def _seg_counts(br):
    return pl.pallas_call(
        _cnt_body,
        grid=(_NB,),
        in_specs=[pl.BlockSpec((1, 1, _BLK), lambda i: (i, 0, 0))],
        out_specs=pl.BlockSpec((_G, _H), lambda i: (0, 0)),
        out_shape=jax.ShapeDtypeStruct((_G, _H), _F32),
    )(br)


def _hv_body(h_ref, bc_ref, v_ref, hv_ref):
    oh = (bc_ref[...] == lax.broadcasted_iota(_F32, (1, _G), 1)).astype(_F32)
    hv_ref[...] = h_ref[...] + jnp.dot(oh, v_ref[...],
                                       preferred_element_type=_F32)


def _add_vnode(h, bc, v):
    return pl.pallas_call(
        _hv_body,
        grid=(_NB,),
        in_specs=[
            pl.BlockSpec((_BLK, _H), lambda i: (i, 0)),
            pl.BlockSpec((_BLK, 1), lambda i: (i, 0)),
            pl.BlockSpec((_G, _H), lambda i: (0, 0)),
        ],
        out_specs=pl.BlockSpec((_BLK, _H), lambda i: (i, 0)),
        out_shape=jax.ShapeDtypeStruct((_N, _H), _F32),
    )(h, bc, v)


def _zmlp_body(hv_ref, agg_ref, br_ref, w1_ref, b1_ref, w2_ref, b2_ref,
               z_ref, s1_ref, s2_ref):
    i = pl.program_id(0)
    z0 = hv_ref[...] + agg_ref[0] + agg_ref[1]
    t = jnp.maximum(
        jnp.dot(z0, w1_ref[...], preferred_element_type=_F32) + b1_ref[...],
        0.0)
    z = jnp.dot(t, w2_ref[...], preferred_element_type=_F32) + b2_ref[...]
    z_ref[...] = z
    oht = (br_ref[0] == lax.broadcasted_iota(_F32, (_G, 1), 0)).astype(_F32)
    s1 = jnp.dot(oht, z, preferred_element_type=_F32)
    s2 = jnp.dot(oht, z * z, preferred_element_type=_F32)

    @pl.when(i == 0)
    def _():
        s1_ref[...] = jnp.zeros_like(s1_ref)
        s2_ref[...] = jnp.zeros_like(s2_ref)

    s1_ref[...] += s1
    s2_ref[...] += s2


def _node_mlp(hv, agg2, br, w1t, b1, w2t, b2):
    return pl.pallas_call(
        _zmlp_body,
        grid=(_NB,),
        in_specs=[
            pl.BlockSpec((_BLK, _H), lambda i: (i, 0)),
            pl.BlockSpec((2, _BLK, _H), lambda i: (0, i, 0)),
            pl.BlockSpec((1, 1, _BLK), lambda i: (i, 0, 0)),
            pl.BlockSpec((_H, _H), lambda i: (0, 0)),
            pl.BlockSpec((1, _H), lambda i: (0, 0)),
            pl.BlockSpec((_H, _H), lambda i: (0, 0)),
            pl.BlockSpec((1, _H), lambda i: (0, 0)),
        ],
        out_specs=[
            pl.BlockSpec((_BLK, _H), lambda i: (i, 0)),
            pl.BlockSpec((_G, _H), lambda i: (0, 0)),
            pl.BlockSpec((_G, _H), lambda i: (0, 0)),
        ],
        out_shape=[
            jax.ShapeDtypeStruct((_N, _H), _F32),
            jax.ShapeDtypeStruct((_G, _H), _F32),
            jax.ShapeDtypeStruct((_G, _H), _F32),
        ],
    )(hv, agg2, br, w1t, b1, w2t, b2)


def _norm_body(z_ref, hv_ref, bc_ref, br_ref, s1_ref, s2_ref, cnt_ref,
               g_ref, be_ref, al_ref, hn_ref, p_ref):
    i = pl.program_id(0)
    cnt = jnp.maximum(cnt_ref[...], 1.0)
    s1 = s1_ref[...]
    s2 = s2_ref[...]
    al = al_ref[...]
    am = al * (s1 / cnt)
    ssq = s2 - 2.0 * am * s1 + cnt * am * am
    inv = lax.rsqrt(ssq / cnt + _EPS)
    oh = (bc_ref[...] == lax.broadcasted_iota(_F32, (1, _G), 1)).astype(_F32)
    amb = jnp.dot(oh, am, preferred_element_type=_F32)
    invb = jnp.dot(oh, inv, preferred_element_type=_F32)
    out = z_ref[...] - amb
    gn = g_ref[...] * out * invb + be_ref[...]
    hn = jnp.maximum(gn, 0.0) + hv_ref[...]
    hn_ref[...] = hn
    oht = (br_ref[0] == lax.broadcasted_iota(_F32, (_G, 1), 0)).astype(_F32)
    part = jnp.dot(oht, hn, preferred_element_type=_F32)

    @pl.when(i == 0)
    def _():
        p_ref[...] = jnp.zeros_like(p_ref)

    p_ref[...] += part


def _graph_norm(z, hv, bc, br, s1, s2, cnt, gamma, beta, alpha):
    return pl.pallas_call(
        _norm_body,
        grid=(_NB,),
        in_specs=[
            pl.BlockSpec((_BLK, _H), lambda i: (i, 0)),
            pl.BlockSpec((_BLK, _H), lambda i: (i, 0)),
            pl.BlockSpec((_BLK, 1), lambda i: (i, 0)),
            pl.BlockSpec((1, 1, _BLK), lambda i: (i, 0, 0)),
            pl.BlockSpec((_G, _H), lambda i: (0, 0)),
            pl.BlockSpec((_G, _H), lambda i: (0, 0)),
            pl.BlockSpec((_G, _H), lambda i: (0, 0)),
            pl.BlockSpec((1, _H), lambda i: (0, 0)),
            pl.BlockSpec((1, _H), lambda i: (0, 0)),
            pl.BlockSpec((1, _H), lambda i: (0, 0)),
        ],
        out_specs=[
            pl.BlockSpec((_BLK, _H), lambda i: (i, 0)),
            pl.BlockSpec((_G, _H), lambda i: (0, 0)),
        ],
        out_shape=[
            jax.ShapeDtypeStruct((_N, _H), _F32),
            jax.ShapeDtypeStruct((_G, _H), _F32),
        ],
    )(z, hv, bc, br, s1, s2, cnt, gamma, beta, alpha)


def _vupd_body(v_ref, p_ref, w1_ref, b1_ref, w2_ref, b2_ref, vo_ref):
    t = jnp.maximum(
        jnp.dot(p_ref[...], w1_ref[...], preferred_element_type=_F32)
        + b1_ref[...], 0.0)
    vo_ref[...] = (v_ref[...]
                   + jnp.dot(t, w2_ref[...], preferred_element_type=_F32)
                   + b2_ref[...])


def _vnode_update(v, pooled, w1t, b1, w2t, b2):
    full = pl.BlockSpec((_G, _H), lambda: (0, 0))
    row = pl.BlockSpec((1, _H), lambda: (0, 0))
    sq = pl.BlockSpec((_H, _H), lambda: (0, 0))
    return pl.pallas_call(
        _vupd_body,
        grid=(),
        in_specs=[full, full, sq, row, sq, row],
        out_specs=full,
        out_shape=jax.ShapeDtypeStruct((_G, _H), _F32),
    )(v, pooled, w1t, b1, w2t, b2)


def _head_body(h0_ref, h1_ref, h2_ref, h3_ref, h4_ref, h5_ref,
               w1s_ref, b1_ref, w2_ref, b2_ref, o_ref):
    hs = (h0_ref, h1_ref, h2_ref, h3_ref, h4_ref, h5_ref)
    acc = jnp.dot(hs[0][...], w1s_ref[0], preferred_element_type=_F32)
    for l in range(1, _L):
        acc += jnp.dot(hs[l][...], w1s_ref[l], preferred_element_type=_F32)
    t = jnp.maximum(acc + b1_ref[...], 0.0)
    o_ref[...] = jnp.dot(t, w2_ref[...], preferred_element_type=_F32) \
        + b2_ref[...]


def _head(outs, w1s, b1, w2p, b2p):
    blk = pl.BlockSpec((_BLK, _H), lambda i: (i, 0))
    return pl.pallas_call(
        _head_body,
        grid=(_NB,),
        in_specs=[blk] * _L + [
            pl.BlockSpec((_L, _H, _H), lambda i: (0, 0, 0)),
            pl.BlockSpec((1, _H), lambda i: (0, 0)),
            pl.BlockSpec((_H, 8), lambda i: (0, 0)),
            pl.BlockSpec((1, 8), lambda i: (0, 0)),
        ],
        out_specs=pl.BlockSpec((_BLK, 8), lambda i: (i, 0)),
        out_shape=jax.ShapeDtypeStruct((_N, 8), _F32),
    )(*outs, w1s, b1, w2p, b2p)


def _sc_agg(hv, e_rows, src, dst):
    mesh = plsc.VectorSubcoreMesh(core_axis_name="c", subcore_axis_name="s")

    @functools.partial(
        pl.kernel,
        out_type=jax.ShapeDtypeStruct((2, _N, _H), _F32),
        mesh=mesh,
        scratch_types=[
            pltpu.VMEM((_CH,), jnp.int32),
            pltpu.VMEM((_CH,), jnp.int32),
            pltpu.VMEM((_CH, _H), _F32),
            pltpu.VMEM((_CH, _H), _F32),
            pltpu.VMEM((_ZR, _H), _F32),
            pltpu.VMEM_SHARED((_N, _H), _F32),
            pltpu.SemaphoreType.DMA,
        ],
    )
    def agg_kernel(hv_hbm, e_hbm, src_hbm, dst_hbm, out_hbm,
                   sidx, didx, gb, eb, zb, acc, sem):
        cid = lax.axis_index("c")
        sid = lax.axis_index("s")
        zero = jnp.zeros((16,), _F32)

        @pl.loop(0, _ZR)
        def _(i):
            for j in range(_H // 16):
                zb[i, pl.ds(j * 16, 16)] = zero

        @pl.loop(0, _RPS // _ZR)
        def _(k):
            pltpu.sync_copy(zb, acc.at[pl.ds(sid * _RPS + k * _ZR, _ZR), :])

        plsc.subcore_barrier()

        tile_base = (cid * 16 + sid) * _EPT

        @pl.loop(0, _NCH)
        def _(t):
            base = tile_base + t * _CH
            pltpu.sync_copy(src_hbm.at[pl.ds(base, _CH)], sidx)
            pltpu.sync_copy(dst_hbm.at[pl.ds(base, _CH)], didx)
            pltpu.sync_copy(e_hbm.at[pl.ds(base, _CH), :], eb)
            pltpu.async_copy(hv_hbm.at[sidx], gb, sem).wait()

            @pl.loop(0, _CH)
            def _(i):
                for j in range(_H // 16):
                    sl = pl.ds(j * 16, 16)
                    gb[i, sl] = jnp.maximum(gb[i, sl] + eb[i, sl], 0.0)

            pltpu.sync_copy(gb, acc.at[didx], add=True)

        plsc.subcore_barrier()

        @pl.loop(0, _RPS // _ZR)
        def _(k):
            r0 = sid * _RPS + k * _ZR
            pltpu.sync_copy(acc.at[pl.ds(r0, _ZR), :],
                            out_hbm.at[cid, pl.ds(r0, _ZR), :])

    return agg_kernel(hv, e_rows, src, dst)


def kernel(x, edge_index, edge_attr, batch, params):
    src = edge_index[0]
    dst = edge_index[1]
    bc = batch.astype(_F32).reshape(_N, 1)
    br = batch.astype(_F32).reshape(_NB, 1, _BLK)

    cnt = _seg_counts(br)
    e_rows = [
        _edge_mlp(edge_attr, c["We"].T, c["be"].reshape(1, _H))
        for c in params["convs"]
    ]

    pv = params["vmlp"]
    wv1t = pv["W1"].T
    bv1 = pv["b1"].reshape(1, _H)
    wv2t = pv["W2"].T
    bv2 = pv["b2"].reshape(1, _H)

    v = jnp.broadcast_to(params["vnode_emb"], (_G, _H))
    h = x
    outs = []
    pooled = None
    for l in range(_L):
        if l > 0:
            v = _vnode_update(v, pooled, wv1t, bv1, wv2t, bv2)
        hv = _add_vnode(h, bc, v)
        agg2 = _sc_agg(hv, e_rows[l], src, dst)
        c = params["convs"][l]
        z, s1, s2 = _node_mlp(hv, agg2, br, c["W1"].T, c["b1"].reshape(1, _H),
                              c["W2"].T, c["b2"].reshape(1, _H))
        nrm = params["norms"][l]
        hn, pooled = _graph_norm(
            z, hv, bc, br, s1, s2, cnt,
            nrm["gamma"].reshape(1, _H), nrm["beta"].reshape(1, _H),
            nrm["alpha"].reshape(1, _H))
        outs.append(hn)
        h = hn

    ph = params["head"]
    w1s = ph["W1"].T.reshape(_L, _H, _H)
    b1 = ph["b1"].reshape(1, _H)
    w2p = jnp.pad(ph["W2"].T, ((0, 0), (0, 8 - _OUT)))
    b2p = jnp.pad(ph["b2"].reshape(1, _OUT), ((0, 0), (0, 8 - _OUT)))
    head_out = _head(outs, w1s, b1, w2p, b2p)
    return head_out[:, :_SPLIT], head_out[:, _SPLIT:_OUT]


# speedup vs baseline: 2.7246x; 2.7246x over previous
import functools

import jax
import jax.numpy as jnp
from jax import lax
from jax.experimental import pallas as pl
from jax.experimental.pallas import tpu as pltpu
from jax.experimental.pallas import tpu_sc as plsc

_N = 10000
_E = 320000
_DE = 16
_H = 128
_L = 6
_OUT = 6
_SPLIT = 3
_G = 8
_EPS = 1e-5

_BLK = 2000
_NB = _N // _BLK
_EB = 2560
_NTILES = 32
_EPT = _E // _NTILES
_CH = 80
_NCH = _EPT // _CH
_RCH = 80
_NRC = _N // _RCH

_F32 = jnp.float32


def _edge_mlp_body(ea_ref, w_ref, b_ref, out_ref):
    out_ref[...] = (
        jnp.dot(ea_ref[...], w_ref[...], preferred_element_type=_F32)
        + b_ref[...]
    )


def _edge_mlp(edge_attr, wt, b_row):
    return pl.pallas_call(
        _edge_mlp_body,
        grid=(_E // _EB,),
        in_specs=[
            pl.BlockSpec((_EB, _DE), lambda i: (i, 0)),
            pl.BlockSpec((_DE, _H), lambda i: (0, 0)),
            pl.BlockSpec((1, _H), lambda i: (0, 0)),
        ],
        out_specs=pl.BlockSpec((_EB, _H), lambda i: (i, 0)),
        out_shape=jax.ShapeDtypeStruct((_E, _H), _F32),
    )(edge_attr, wt, b_row)


def _cnt_body(br_ref, cnt_ref):
    i = pl.program_id(0)
    oht = (br_ref[0] == lax.broadcasted_iota(jnp.int32, (_G, 1), 0).astype(_F32)).astype(_F32)
    part = jnp.broadcast_to(jnp.sum(oht, axis=1, keepdims=True), (_G, _H))

    @pl.when(i == 0)
    def _():
        cnt_ref[...] = jnp.zeros_like(cnt_ref)

    cnt_ref[...] += part


def _seg_counts(br):
    return pl.pallas_call(
        _cnt_body,
        grid=(_NB,),
        in_specs=[pl.BlockSpec((1, 1, _BLK), lambda i: (i, 0, 0))],
        out_specs=pl.BlockSpec((_G, _H), lambda i: (0, 0)),
        out_shape=jax.ShapeDtypeStruct((_G, _H), _F32),
    )(br)


def _seg_sum_rows(bm, x):
    rows = []
    for g in range(_G):
        m = bm == float(g)
        rows.append(jnp.sum(jnp.where(m, x, 0.0), axis=0, keepdims=True))
    return jnp.concatenate(rows, axis=0)


def _seg_gather_rows(bm, tab):
    acc = jnp.where(bm == 0.0, tab[0], 0.0)
    for g in range(1, _G):
        acc += jnp.where(bm == float(g), tab[g], 0.0)
    return acc


def _hv_body(h_ref, bc_ref, v_ref, hv_ref):
    hv_ref[...] = h_ref[...] + _seg_gather_rows(bc_ref[...], v_ref[...])


def _add_vnode(h, bc, v):
    return pl.pallas_call(
        _hv_body,
        grid=(_NB,),
        in_specs=[
            pl.BlockSpec((_BLK, _H), lambda i: (i, 0)),
            pl.BlockSpec((_BLK, 1), lambda i: (i, 0)),
            pl.BlockSpec((_G, _H), lambda i: (0, 0)),
        ],
        out_specs=pl.BlockSpec((_BLK, _H), lambda i: (i, 0)),
        out_shape=jax.ShapeDtypeStruct((_N, _H), _F32),
    )(h, bc, v)


def _zmlp_body(hv_ref, agg_ref, bc_ref, w1_ref, b1_ref, w2_ref, b2_ref,
               z_ref, s1_ref):
    i = pl.program_id(0)
    z0 = hv_ref[...] + agg_ref[0] + agg_ref[1]
    t = jnp.maximum(
        jnp.dot(z0, w1_ref[...], preferred_element_type=_F32) + b1_ref[...],
        0.0)
    z = jnp.dot(t, w2_ref[...], preferred_element_type=_F32) + b2_ref[...]
    z_ref[...] = z
    s1 = _seg_sum_rows(bc_ref[...], z)

    @pl.when(i == 0)
    def _():
        s1_ref[...] = jnp.zeros_like(s1_ref)

    s1_ref[...] += s1


def _node_mlp(hv, agg2, bc, w1t, b1, w2t, b2):
    return pl.pallas_call(
        _zmlp_body,
        grid=(_NB,),
        in_specs=[
            pl.BlockSpec((_BLK, _H), lambda i: (i, 0)),
            pl.BlockSpec((2, _BLK, _H), lambda i: (0, i, 0)),
            pl.BlockSpec((_BLK, 1), lambda i: (i, 0)),
            pl.BlockSpec((_H, _H), lambda i: (0, 0)),
            pl.BlockSpec((1, _H), lambda i: (0, 0)),
            pl.BlockSpec((_H, _H), lambda i: (0, 0)),
            pl.BlockSpec((1, _H), lambda i: (0, 0)),
        ],
        out_specs=[
            pl.BlockSpec((_BLK, _H), lambda i: (i, 0)),
            pl.BlockSpec((_G, _H), lambda i: (0, 0)),
        ],
        out_shape=[
            jax.ShapeDtypeStruct((_N, _H), _F32),
            jax.ShapeDtypeStruct((_G, _H), _F32),
        ],
    )(hv, agg2, bc, w1t, b1, w2t, b2)


def _ssq_body(z_ref, bc_ref, s1_ref, cnt_ref, al_ref, out_ref, q_ref):
    i = pl.program_id(0)
    cnt = jnp.maximum(cnt_ref[...], 1.0)
    am = al_ref[...] * (s1_ref[...] / cnt)
    bm = bc_ref[...]
    out = z_ref[...] - _seg_gather_rows(bm, am)
    out_ref[...] = out
    q = _seg_sum_rows(bm, out * out)

    @pl.when(i == 0)
    def _():
        q_ref[...] = jnp.zeros_like(q_ref)

    q_ref[...] += q


def _center(z, bc, s1, cnt, alpha):
    return pl.pallas_call(
        _ssq_body,
        grid=(_NB,),
        in_specs=[
            pl.BlockSpec((_BLK, _H), lambda i: (i, 0)),
            pl.BlockSpec((_BLK, 1), lambda i: (i, 0)),
            pl.BlockSpec((_G, _H), lambda i: (0, 0)),
            pl.BlockSpec((_G, _H), lambda i: (0, 0)),
            pl.BlockSpec((1, _H), lambda i: (0, 0)),
        ],
        out_specs=[
            pl.BlockSpec((_BLK, _H), lambda i: (i, 0)),
            pl.BlockSpec((_G, _H), lambda i: (0, 0)),
        ],
        out_shape=[
            jax.ShapeDtypeStruct((_N, _H), _F32),
            jax.ShapeDtypeStruct((_G, _H), _F32),
        ],
    )(z, bc, s1, cnt, alpha)


def _norm_body(o_ref, hv_ref, bc_ref, q_ref, cnt_ref,
               g_ref, be_ref, hn_ref, p_ref):
    i = pl.program_id(0)
    cnt = jnp.maximum(cnt_ref[...], 1.0)
    inv = 1.0 / jnp.sqrt(q_ref[...] / cnt + _EPS)
    bm = bc_ref[...]
    invb = _seg_gather_rows(bm, inv)
    gn = g_ref[...] * o_ref[...] * invb + be_ref[...]
    hn = jnp.maximum(gn, 0.0) + hv_ref[...]
    hn_ref[...] = hn
    part = _seg_sum_rows(bm, hn)

    @pl.when(i == 0)
    def _():
        p_ref[...] = jnp.zeros_like(p_ref)

    p_ref[...] += part


def _graph_norm(out, hv, bc, q, cnt, gamma, beta):
    return pl.pallas_call(
        _norm_body,
        grid=(_NB,),
        in_specs=[
            pl.BlockSpec((_BLK, _H), lambda i: (i, 0)),
            pl.BlockSpec((_BLK, _H), lambda i: (i, 0)),
            pl.BlockSpec((_BLK, 1), lambda i: (i, 0)),
            pl.BlockSpec((_G, _H), lambda i: (0, 0)),
            pl.BlockSpec((_G, _H), lambda i: (0, 0)),
            pl.BlockSpec((1, _H), lambda i: (0, 0)),
            pl.BlockSpec((1, _H), lambda i: (0, 0)),
        ],
        out_specs=[
            pl.BlockSpec((_BLK, _H), lambda i: (i, 0)),
            pl.BlockSpec((_G, _H), lambda i: (0, 0)),
        ],
        out_shape=[
            jax.ShapeDtypeStruct((_N, _H), _F32),
            jax.ShapeDtypeStruct((_G, _H), _F32),
        ],
    )(out, hv, bc, q, cnt, gamma, beta)


def _vupd_body(v_ref, p_ref, w1_ref, b1_ref, w2_ref, b2_ref, vo_ref):
    t = jnp.maximum(
        jnp.dot(p_ref[...], w1_ref[...], preferred_element_type=_F32)
        + b1_ref[...], 0.0)
    vo_ref[...] = (v_ref[...]
                   + jnp.dot(t, w2_ref[...], preferred_element_type=_F32)
                   + b2_ref[...])


def _vnode_update(v, pooled, w1t, b1, w2t, b2):
    full = pl.BlockSpec((_G, _H), lambda: (0, 0))
    row = pl.BlockSpec((1, _H), lambda: (0, 0))
    sq = pl.BlockSpec((_H, _H), lambda: (0, 0))
    return pl.pallas_call(
        _vupd_body,
        grid=(),
        in_specs=[full, full, sq, row, sq, row],
        out_specs=full,
        out_shape=jax.ShapeDtypeStruct((_G, _H), _F32),
    )(v, pooled, w1t, b1, w2t, b2)


def _head_body(h0_ref, h1_ref, h2_ref, h3_ref, h4_ref, h5_ref,
               w1s_ref, b1_ref, w2_ref, b2_ref, o_ref):
    hs = (h0_ref, h1_ref, h2_ref, h3_ref, h4_ref, h5_ref)
    acc = jnp.dot(hs[0][...], w1s_ref[0], preferred_element_type=_F32)
    for l in range(1, _L):
        acc += jnp.dot(hs[l][...], w1s_ref[l], preferred_element_type=_F32)
    t = jnp.maximum(acc + b1_ref[...], 0.0)
    o_ref[...] = jnp.dot(t, w2_ref[...], preferred_element_type=_F32) \
        + b2_ref[...]


def _head(outs, w1s, b1, w2p, b2p):
    blk = pl.BlockSpec((_BLK, _H), lambda i: (i, 0))
    return pl.pallas_call(
        _head_body,
        grid=(_NB,),
        in_specs=[blk] * _L + [
            pl.BlockSpec((_L, _H, _H), lambda i: (0, 0, 0)),
            pl.BlockSpec((1, _H), lambda i: (0, 0)),
            pl.BlockSpec((_H, 8), lambda i: (0, 0)),
            pl.BlockSpec((1, 8), lambda i: (0, 0)),
        ],
        out_specs=pl.BlockSpec((_BLK, 8), lambda i: (i, 0)),
        out_shape=jax.ShapeDtypeStruct((_N, 8), _F32),
    )(*outs, w1s, b1, w2p, b2p)


def _sc_agg(hv, e_rows, src, dst):
    mesh = plsc.VectorSubcoreMesh(core_axis_name="c", subcore_axis_name="s")

    @functools.partial(
        pl.kernel,
        out_type=jax.ShapeDtypeStruct((2, _N, _H), _F32),
        mesh=mesh,
        scratch_types=[
            pltpu.VMEM((_CH,), jnp.int32),
            pltpu.VMEM((_CH,), jnp.int32),
            pltpu.VMEM((_CH, _H), _F32),
            pltpu.VMEM((_CH, _H), _F32),
            pltpu.VMEM((_RCH, _H), _F32),
            pltpu.VMEM_SHARED((_N, _H), _F32),
            pltpu.SemaphoreType.DMA,
        ],
    )
    def agg_kernel(hv_hbm, e_hbm, src_hbm, dst_hbm, out_hbm,
                   sidx, didx, gb, eb, zb, acc, sem):
        cid = lax.axis_index("c")
        sid = lax.axis_index("s")
        zero = jnp.zeros((16,), _F32)

        @pl.loop(0, _RCH)
        def _(i):
            for j in range(_H // 16):
                zb[i, pl.ds(j * 16, 16)] = zero

        @pl.loop(0, (_NRC + 15) // 16)
        def _(k):
            c = k * 16 + sid

            @pl.when(c < _NRC)
            def _():
                pltpu.sync_copy(zb, acc.at[pl.ds(c * _RCH, _RCH), :])

        plsc.subcore_barrier()

        tile_base = (cid * 16 + sid) * _EPT

        @pl.loop(0, _NCH)
        def _(t):
            base = tile_base + t * _CH
            pltpu.sync_copy(src_hbm.at[pl.ds(base, _CH)], sidx)
            pltpu.sync_copy(dst_hbm.at[pl.ds(base, _CH)], didx)
            pltpu.sync_copy(e_hbm.at[pl.ds(base, _CH), :], eb)
            pltpu.async_copy(hv_hbm.at[sidx], gb, sem).wait()

            @pl.loop(0, _CH)
            def _(i):
                for j in range(_H // 16):
                    sl = pl.ds(j * 16, 16)
                    gb[i, sl] = jnp.maximum(gb[i, sl] + eb[i, sl], 0.0)

            pltpu.sync_copy(gb, acc.at[didx], add=True)

        plsc.subcore_barrier()

        @pl.loop(0, (_NRC + 15) // 16)
        def _(k):
            c = k * 16 + sid

            @pl.when(c < _NRC)
            def _():
                r0 = c * _RCH
                pltpu.sync_copy(acc.at[pl.ds(r0, _RCH), :],
                                out_hbm.at[cid, pl.ds(r0, _RCH), :])

    return agg_kernel(hv, e_rows, src, dst)


def kernel(x, edge_index, edge_attr, batch, params):
    src = edge_index[0]
    dst = edge_index[1]
    bc = batch.astype(_F32).reshape(_N, 1)
    br = batch.astype(_F32).reshape(_NB, 1, _BLK)

    cnt = _seg_counts(br)
    e_rows = [
        _edge_mlp(edge_attr, c["We"].T, c["be"].reshape(1, _H))
        for c in params["convs"]
    ]

    pv = params["vmlp"]
    wv1t = pv["W1"].T
    bv1 = pv["b1"].reshape(1, _H)
    wv2t = pv["W2"].T
    bv2 = pv["b2"].reshape(1, _H)

    v = jnp.broadcast_to(params["vnode_emb"], (_G, _H))
    h = x
    outs = []
    pooled = None
    for l in range(_L):
        if l > 0:
            v = _vnode_update(v, pooled, wv1t, bv1, wv2t, bv2)
        hv = _add_vnode(h, bc, v)
        agg2 = _sc_agg(hv, e_rows[l], src, dst)
        c = params["convs"][l]
        z, s1 = _node_mlp(hv, agg2, bc, c["W1"].T, c["b1"].reshape(1, _H),
                          c["W2"].T, c["b2"].reshape(1, _H))
        nrm = params["norms"][l]
        ctr, q = _center(z, bc, s1, cnt, nrm["alpha"].reshape(1, _H))
        hn, pooled = _graph_norm(
            ctr, hv, bc, q, cnt,
            nrm["gamma"].reshape(1, _H), nrm["beta"].reshape(1, _H))
        outs.append(hn)
        h = hn

    ph = params["head"]
    w1s = ph["W1"].T.reshape(_L, _H, _H)
    b1 = ph["b1"].reshape(1, _H)
    w2p = jnp.pad(ph["W2"].T, ((0, 0), (0, 8 - _OUT)))
    b2p = jnp.pad(ph["b2"].reshape(1, _OUT), ((0, 0), (0, 8 - _OUT)))
    head_out = _head(outs, w1s, b1, w2p, b2p)
    return head_out[:, :_SPLIT], head_out[:, _SPLIT:_OUT]


# speedup vs baseline: 4.1979x; 1.5407x over previous
import functools

import jax
import jax.numpy as jnp
from jax import lax
from jax.experimental import pallas as pl
from jax.experimental.pallas import tpu as pltpu
from jax.experimental.pallas import tpu_sc as plsc

_N = 10000
_E = 320000
_DE = 16
_H = 128
_L = 6
_OUT = 6
_SPLIT = 3
_G = 8
_EPS = 1e-5

_BLK = 2000
_NB = _N // _BLK
_EB = 2560
_NTILES = 32
_EPT = _E // _NTILES
_CH = 80
_NCH = _EPT // _CH
_RCH = 80
_NRC = _N // _RCH

_F32 = jnp.float32


def _edge_mlp_body(ea_ref, w_ref, b_ref, out_ref):
    out_ref[...] = (
        jnp.dot(ea_ref[...], w_ref[...], preferred_element_type=_F32)
        + b_ref[...]
    )


def _edge_mlp(edge_attr, wt, b_row):
    return pl.pallas_call(
        _edge_mlp_body,
        grid=(_E // _EB,),
        in_specs=[
            pl.BlockSpec((_EB, _DE), lambda i: (i, 0)),
            pl.BlockSpec((_DE, _H), lambda i: (0, 0)),
            pl.BlockSpec((1, _H), lambda i: (0, 0)),
        ],
        out_specs=pl.BlockSpec((_EB, _H), lambda i: (i, 0)),
        out_shape=jax.ShapeDtypeStruct((_E, _H), _F32),
    )(edge_attr, wt, b_row)


def _cnt_body(br_ref, cnt_ref):
    i = pl.program_id(0)
    oht = (br_ref[0] == lax.broadcasted_iota(jnp.int32, (_G, 1), 0).astype(_F32)).astype(_F32)
    part = jnp.broadcast_to(jnp.sum(oht, axis=1, keepdims=True), (_G, _H))

    @pl.when(i == 0)
    def _():
        cnt_ref[...] = jnp.zeros_like(cnt_ref)

    cnt_ref[...] += part


def _seg_counts(br):
    return pl.pallas_call(
        _cnt_body,
        grid=(_NB,),
        in_specs=[pl.BlockSpec((1, 1, _BLK), lambda i: (i, 0, 0))],
        out_specs=pl.BlockSpec((_G, _H), lambda i: (0, 0)),
        out_shape=jax.ShapeDtypeStruct((_G, _H), _F32),
    )(br)


def _seg_sum_rows(bm, x):
    rows = []
    for g in range(_G):
        m = bm == float(g)
        rows.append(jnp.sum(jnp.where(m, x, 0.0), axis=0, keepdims=True))
    return jnp.concatenate(rows, axis=0)


def _seg_gather_rows(bm, tab):
    acc = jnp.where(bm == 0.0, tab[0], 0.0)
    for g in range(1, _G):
        acc += jnp.where(bm == float(g), tab[g], 0.0)
    return acc


def _hv_body(h_ref, bc_ref, v_ref, hv_ref):
    hv_ref[...] = h_ref[...] + _seg_gather_rows(bc_ref[...], v_ref[...])


def _add_vnode(h, bc, v):
    return pl.pallas_call(
        _hv_body,
        grid=(_NB,),
        in_specs=[
            pl.BlockSpec((_BLK, _H), lambda i: (i, 0)),
            pl.BlockSpec((_BLK, 1), lambda i: (i, 0)),
            pl.BlockSpec((_G, _H), lambda i: (0, 0)),
        ],
        out_specs=pl.BlockSpec((_BLK, _H), lambda i: (i, 0)),
        out_shape=jax.ShapeDtypeStruct((_N, _H), _F32),
    )(h, bc, v)


def _zmlp_body(hv_ref, agg_ref, bc_ref, w1_ref, b1_ref, w2_ref, b2_ref,
               z_ref, s1_ref):
    i = pl.program_id(0)
    z0 = hv_ref[...] + agg_ref[0] + agg_ref[1]
    t = jnp.maximum(
        jnp.dot(z0, w1_ref[...], preferred_element_type=_F32) + b1_ref[...],
        0.0)
    z = jnp.dot(t, w2_ref[...], preferred_element_type=_F32) + b2_ref[...]
    z_ref[...] = z
    s1 = _seg_sum_rows(bc_ref[...], z)

    @pl.when(i == 0)
    def _():
        s1_ref[...] = jnp.zeros_like(s1_ref)

    s1_ref[...] += s1


def _node_mlp(hv, agg2, bc, w1t, b1, w2t, b2):
    return pl.pallas_call(
        _zmlp_body,
        grid=(_NB,),
        in_specs=[
            pl.BlockSpec((_BLK, _H), lambda i: (i, 0)),
            pl.BlockSpec((2, _BLK, _H), lambda i: (0, i, 0)),
            pl.BlockSpec((_BLK, 1), lambda i: (i, 0)),
            pl.BlockSpec((_H, _H), lambda i: (0, 0)),
            pl.BlockSpec((1, _H), lambda i: (0, 0)),
            pl.BlockSpec((_H, _H), lambda i: (0, 0)),
            pl.BlockSpec((1, _H), lambda i: (0, 0)),
        ],
        out_specs=[
            pl.BlockSpec((_BLK, _H), lambda i: (i, 0)),
            pl.BlockSpec((_G, _H), lambda i: (0, 0)),
        ],
        out_shape=[
            jax.ShapeDtypeStruct((_N, _H), _F32),
            jax.ShapeDtypeStruct((_G, _H), _F32),
        ],
    )(hv, agg2, bc, w1t, b1, w2t, b2)


def _ssq_body(z_ref, bc_ref, s1_ref, cnt_ref, al_ref, out_ref, q_ref):
    i = pl.program_id(0)
    cnt = jnp.maximum(cnt_ref[...], 1.0)
    am = al_ref[...] * (s1_ref[...] / cnt)
    bm = bc_ref[...]
    out = z_ref[...] - _seg_gather_rows(bm, am)
    out_ref[...] = out
    q = _seg_sum_rows(bm, out * out)

    @pl.when(i == 0)
    def _():
        q_ref[...] = jnp.zeros_like(q_ref)

    q_ref[...] += q


def _center(z, bc, s1, cnt, alpha):
    return pl.pallas_call(
        _ssq_body,
        grid=(_NB,),
        in_specs=[
            pl.BlockSpec((_BLK, _H), lambda i: (i, 0)),
            pl.BlockSpec((_BLK, 1), lambda i: (i, 0)),
            pl.BlockSpec((_G, _H), lambda i: (0, 0)),
            pl.BlockSpec((_G, _H), lambda i: (0, 0)),
            pl.BlockSpec((1, _H), lambda i: (0, 0)),
        ],
        out_specs=[
            pl.BlockSpec((_BLK, _H), lambda i: (i, 0)),
            pl.BlockSpec((_G, _H), lambda i: (0, 0)),
        ],
        out_shape=[
            jax.ShapeDtypeStruct((_N, _H), _F32),
            jax.ShapeDtypeStruct((_G, _H), _F32),
        ],
    )(z, bc, s1, cnt, alpha)


def _norm_body(o_ref, hv_ref, bc_ref, q_ref, cnt_ref,
               g_ref, be_ref, hn_ref, p_ref):
    i = pl.program_id(0)
    cnt = jnp.maximum(cnt_ref[...], 1.0)
    inv = 1.0 / jnp.sqrt(q_ref[...] / cnt + _EPS)
    bm = bc_ref[...]
    invb = _seg_gather_rows(bm, inv)
    gn = g_ref[...] * o_ref[...] * invb + be_ref[...]
    hn = jnp.maximum(gn, 0.0) + hv_ref[...]
    hn_ref[...] = hn
    part = _seg_sum_rows(bm, hn)

    @pl.when(i == 0)
    def _():
        p_ref[...] = jnp.zeros_like(p_ref)

    p_ref[...] += part


def _graph_norm(out, hv, bc, q, cnt, gamma, beta):
    return pl.pallas_call(
        _norm_body,
        grid=(_NB,),
        in_specs=[
            pl.BlockSpec((_BLK, _H), lambda i: (i, 0)),
            pl.BlockSpec((_BLK, _H), lambda i: (i, 0)),
            pl.BlockSpec((_BLK, 1), lambda i: (i, 0)),
            pl.BlockSpec((_G, _H), lambda i: (0, 0)),
            pl.BlockSpec((_G, _H), lambda i: (0, 0)),
            pl.BlockSpec((1, _H), lambda i: (0, 0)),
            pl.BlockSpec((1, _H), lambda i: (0, 0)),
        ],
        out_specs=[
            pl.BlockSpec((_BLK, _H), lambda i: (i, 0)),
            pl.BlockSpec((_G, _H), lambda i: (0, 0)),
        ],
        out_shape=[
            jax.ShapeDtypeStruct((_N, _H), _F32),
            jax.ShapeDtypeStruct((_G, _H), _F32),
        ],
    )(out, hv, bc, q, cnt, gamma, beta)


def _vupd_body(v_ref, p_ref, w1_ref, b1_ref, w2_ref, b2_ref, vo_ref):
    t = jnp.maximum(
        jnp.dot(p_ref[...], w1_ref[...], preferred_element_type=_F32)
        + b1_ref[...], 0.0)
    vo_ref[...] = (v_ref[...]
                   + jnp.dot(t, w2_ref[...], preferred_element_type=_F32)
                   + b2_ref[...])


def _vnode_update(v, pooled, w1t, b1, w2t, b2):
    full = pl.BlockSpec((_G, _H), lambda: (0, 0))
    row = pl.BlockSpec((1, _H), lambda: (0, 0))
    sq = pl.BlockSpec((_H, _H), lambda: (0, 0))
    return pl.pallas_call(
        _vupd_body,
        grid=(),
        in_specs=[full, full, sq, row, sq, row],
        out_specs=full,
        out_shape=jax.ShapeDtypeStruct((_G, _H), _F32),
    )(v, pooled, w1t, b1, w2t, b2)


def _head_body(h0_ref, h1_ref, h2_ref, h3_ref, h4_ref, h5_ref,
               w1s_ref, b1_ref, w2_ref, b2_ref, o_ref):
    hs = (h0_ref, h1_ref, h2_ref, h3_ref, h4_ref, h5_ref)
    acc = jnp.dot(hs[0][...], w1s_ref[0], preferred_element_type=_F32)
    for l in range(1, _L):
        acc += jnp.dot(hs[l][...], w1s_ref[l], preferred_element_type=_F32)
    t = jnp.maximum(acc + b1_ref[...], 0.0)
    o_ref[...] = jnp.dot(t, w2_ref[...], preferred_element_type=_F32) \
        + b2_ref[...]


def _head(outs, w1s, b1, w2p, b2p):
    blk = pl.BlockSpec((_BLK, _H), lambda i: (i, 0))
    return pl.pallas_call(
        _head_body,
        grid=(_NB,),
        in_specs=[blk] * _L + [
            pl.BlockSpec((_L, _H, _H), lambda i: (0, 0, 0)),
            pl.BlockSpec((1, _H), lambda i: (0, 0)),
            pl.BlockSpec((_H, 8), lambda i: (0, 0)),
            pl.BlockSpec((1, 8), lambda i: (0, 0)),
        ],
        out_specs=pl.BlockSpec((_BLK, 8), lambda i: (i, 0)),
        out_shape=jax.ShapeDtypeStruct((_N, 8), _F32),
    )(*outs, w1s, b1, w2p, b2p)


def _sc_agg(hv, e_rows, src, dst):
    mesh = plsc.VectorSubcoreMesh(core_axis_name="c", subcore_axis_name="s")

    @functools.partial(
        pl.kernel,
        out_type=jax.ShapeDtypeStruct((2, _N, _H), _F32),
        mesh=mesh,
        scratch_types=[
            pltpu.VMEM((_CH,), jnp.int32),
            pltpu.VMEM((_CH,), jnp.int32),
            pltpu.VMEM((_CH,), jnp.int32),
            pltpu.VMEM((_CH,), jnp.int32),
            pltpu.VMEM((_CH, _H), _F32),
            pltpu.VMEM((_CH, _H), _F32),
            pltpu.VMEM((_CH, _H), _F32),
            pltpu.VMEM((_CH, _H), _F32),
            pltpu.VMEM_SHARED((_N, _H), _F32),
            pltpu.SemaphoreType.DMA,
            pltpu.SemaphoreType.DMA,
            pltpu.SemaphoreType.DMA,
            pltpu.SemaphoreType.DMA,
        ],
    )
    def agg_kernel(hv_hbm, e_hbm, src_hbm, dst_hbm, out_hbm,
                   sidx0, sidx1, didx0, didx1, gb0, gb1, eb0, eb1, acc,
                   sg0, sg1, se0, se1):
        cid = lax.axis_index("c")
        sid = lax.axis_index("s")
        tid = cid * 16 + sid
        tile_base = tid * _EPT
        gb = (gb0, gb1)
        eb = (eb0, eb1)
        sidx = (sidx0, sidx1)
        didx = (didx0, didx1)
        sg = (sg0, sg1)
        se = (se0, se1)
        zero = jnp.zeros((16,), _F32)

        @pl.loop(0, _RCH)
        def _(i):
            for j in range(_H // 16):
                gb0[i, pl.ds(j * 16, 16)] = zero

        @pl.loop(0, (_NRC + 15) // 16)
        def _(k):
            c = k * 16 + sid

            @pl.when(c < _NRC)
            def _():
                pltpu.sync_copy(gb0, acc.at[pl.ds(c * _RCH, _RCH), :])

        plsc.subcore_barrier()

        def issue_idx(c, b):
            base = tile_base + c * _CH
            pltpu.async_copy(e_hbm.at[pl.ds(base, _CH), :], eb[b], se[b])
            pltpu.async_copy(dst_hbm.at[pl.ds(base, _CH)], didx[b], se[b])
            pltpu.async_copy(src_hbm.at[pl.ds(base, _CH)], sidx[b], se[b])

        def drain_idx(c, b):
            base = tile_base + c * _CH
            pltpu.make_async_copy(
                e_hbm.at[pl.ds(base, _CH), :], eb[b], se[b]).wait()
            pltpu.make_async_copy(
                dst_hbm.at[pl.ds(base, _CH)], didx[b], se[b]).wait()
            pltpu.make_async_copy(
                src_hbm.at[pl.ds(base, _CH)], sidx[b], se[b]).wait()

        def issue_gather(b):
            pltpu.async_copy(hv_hbm.at[sidx[b]], gb[b], sg[b])

        def compute_scatter(c, b):
            base = tile_base + c * _CH
            pltpu.make_async_copy(
                e_hbm.at[pl.ds(base, _CH), :], gb[b], sg[b]).wait()

            @pl.loop(0, _CH)
            def _(i):
                for j in range(_H // 16):
                    sl = pl.ds(j * 16, 16)
                    gb[b][i, sl] = jnp.maximum(gb[b][i, sl] + eb[b][i, sl],
                                               0.0)

            pltpu.sync_copy(gb[b], acc.at[didx[b]], add=True)

        def step(c, b):
            @pl.when(c + 1 < _NCH)
            def _():
                drain_idx(c + 1, 1 - b)
                issue_gather(1 - b)

            compute_scatter(c, b)

            @pl.when(c + 2 < _NCH)
            def _():
                issue_idx(c + 2, b)

        issue_idx(0, 0)
        issue_idx(1, 1)
        drain_idx(0, 0)
        issue_gather(0)

        @pl.loop(0, (_NCH - 1) // 2)
        def _(k):
            for b in range(2):
                step(k * 2 + b, b)

        step(_NCH - 1, (_NCH - 1) % 2)

        plsc.subcore_barrier()

        @pl.loop(0, (_NRC + 15) // 16)
        def _(k):
            c = k * 16 + sid

            @pl.when(c < _NRC)
            def _():
                r0 = c * _RCH
                pltpu.sync_copy(acc.at[pl.ds(r0, _RCH), :],
                                out_hbm.at[cid, pl.ds(r0, _RCH), :])

    return agg_kernel(hv, e_rows, src, dst)


def kernel(x, edge_index, edge_attr, batch, params):
    src = edge_index[0]
    dst = edge_index[1]
    bc = batch.astype(_F32).reshape(_N, 1)
    br = batch.astype(_F32).reshape(_NB, 1, _BLK)

    cnt = _seg_counts(br)
    e_rows = [
        _edge_mlp(edge_attr, c["We"].T, c["be"].reshape(1, _H))
        for c in params["convs"]
    ]

    pv = params["vmlp"]
    wv1t = pv["W1"].T
    bv1 = pv["b1"].reshape(1, _H)
    wv2t = pv["W2"].T
    bv2 = pv["b2"].reshape(1, _H)

    v = jnp.broadcast_to(params["vnode_emb"], (_G, _H))
    h = x
    outs = []
    pooled = None
    for l in range(_L):
        if l > 0:
            v = _vnode_update(v, pooled, wv1t, bv1, wv2t, bv2)
        hv = _add_vnode(h, bc, v)
        agg2 = _sc_agg(hv, e_rows[l], src, dst)
        c = params["convs"][l]
        z, s1 = _node_mlp(hv, agg2, bc, c["W1"].T, c["b1"].reshape(1, _H),
                          c["W2"].T, c["b2"].reshape(1, _H))
        nrm = params["norms"][l]
        ctr, q = _center(z, bc, s1, cnt, nrm["alpha"].reshape(1, _H))
        hn, pooled = _graph_norm(
            ctr, hv, bc, q, cnt,
            nrm["gamma"].reshape(1, _H), nrm["beta"].reshape(1, _H))
        outs.append(hn)
        h = hn

    ph = params["head"]
    w1s = ph["W1"].T.reshape(_L, _H, _H)
    b1 = ph["b1"].reshape(1, _H)
    w2p = jnp.pad(ph["W2"].T, ((0, 0), (0, 8 - _OUT)))
    b2p = jnp.pad(ph["b2"].reshape(1, _OUT), ((0, 0), (0, 8 - _OUT)))
    head_out = _head(outs, w1s, b1, w2p, b2p)
    return head_out[:, :_SPLIT], head_out[:, _SPLIT:_OUT]
